# Initial kernel scaffold; baseline (speedup 1.0000x reference)
import jax
import jax.numpy as jnp
from jax.experimental import pallas as pl


def kernel(ego_embed, edge_index, edge_type, relation_embed):
    raise NotImplementedError("write your pallas kernel here")


# speedup vs baseline: 4.2662x; 4.2662x over previous
import functools

import jax
import jax.numpy as jnp
from jax import lax
from jax.experimental import pallas as pl
from jax.experimental.pallas import tpu as pltpu
from jax.experimental.pallas import tpu_sc as plsc

EPS = 1e-7
MAXN = 1.0 - 1e-5
LN2 = 0.6931471805599453
L = 16
NC = 2
NS = 16
K = 80


def _rsqrt(x):
    i = lax.bitcast_convert_type(x, jnp.int32)
    i = 0x5F3759DF - lax.shift_right_logical(i, 1)
    y = lax.bitcast_convert_type(i, jnp.float32)
    for _ in range(3):
        y = y * (1.5 - 0.5 * x * y * y)
    return y


def _sqrt_pos(x):
    return x * _rsqrt(x)


def _tanh_pos(x):
    return 1.0 - 2.0 / (jnp.exp(2.0 * x) + 1.0)


def _ln(z):
    bits = lax.bitcast_convert_type(z, jnp.int32)
    e = lax.shift_right_logical(bits, 23) - 127
    mbits = jnp.bitwise_or(jnp.bitwise_and(bits, 0x007FFFFF), 0x3F800000)
    m = lax.bitcast_convert_type(mbits, jnp.float32)
    big = m > 1.4142135
    m = jnp.where(big, m * 0.5, m)
    e = (e + jnp.where(big, 1, 0)).astype(jnp.float32)
    s = (m - 1.0) / (m + 1.0)
    s2 = s * s
    lnm = 2.0 * s * (1.0 + s2 * (1.0 / 3.0 + s2 * (0.2 + s2 * (1.0 / 7.0 + s2 / 9.0))))
    return e * LN2 + lnm


def _edge_scalars(hh, tt, rr, htd, hrd, trd):
    def nrm(sq):
        return _sqrt_pos(jnp.clip(sq, EPS, None))

    def proj_scale(nsq):
        n = nrm(nsq)
        return jnp.where(n > MAXN, MAXN / n, 1.0)

    nh = nrm(hh)
    u = _tanh_pos(nh) / nh
    cp = u * proj_scale(u * u * hh)
    pp = cp * cp * hh
    clam = jnp.clip(1.0 - pp, EPS, None)
    lam = 2.0 / clam

    nt = nrm(tt)
    bt = _tanh_pos(0.5 * lam * nt) / nt
    y2 = bt * bt * tt
    xy = cp * bt * htd
    den = jnp.maximum(1.0 + 2.0 * xy + pp * y2, EPS)
    m1 = (1.0 + 2.0 * xy + y2) * cp / den
    m2 = (1.0 - pp) * bt / den
    vsq = m1 * m1 * hh + 2.0 * m1 * m2 * htd + m2 * m2 * tt
    s3 = proj_scale(vsq)
    a1 = m1 * s3
    a2 = m2 * s3
    htsq = s3 * s3 * vsq

    nr = nrm(rr)
    gm = _tanh_pos(0.5 * lam * nr) / nr
    y2r = gm * gm * rr
    xyr = cp * gm * hrd
    denr = jnp.maximum(1.0 + 2.0 * xyr + pp * y2r, EPS)
    k1 = (1.0 + 2.0 * xyr + y2r) * cp / denr
    k2 = (1.0 - pp) * gm / denr
    wsq = k1 * k1 * hh + 2.0 * k1 * k2 * hrd + k2 * k2 * rr
    s4 = proj_scale(wsq)
    b1 = k1 * s4
    b2 = k2 * s4
    hrsq = s4 * s4 * wsq

    xy5 = a1 * b1 * hh + a1 * b2 * hrd + a2 * b1 * htd + a2 * b2 * trd
    den5 = jnp.maximum(1.0 + 2.0 * xy5 + htsq * hrsq, EPS)
    cx = (1.0 + 2.0 * xy5 + hrsq) / den5
    cy = (1.0 - htsq) / den5
    g1 = cx * a1 + cy * b1
    g2 = cx * a2
    g3 = cy * b2
    vsq5 = (g1 * g1 * hh + g2 * g2 * tt + g3 * g3 * rr
            + 2.0 * g1 * g2 * htd + 2.0 * g1 * g3 * hrd + 2.0 * g2 * g3 * trd)
    s5 = proj_scale(vsq5)
    g1 = g1 * s5
    g2 = g2 * s5
    g3 = g3 * s5
    ressq = s5 * s5 * vsq5

    xy6 = -cp * (g1 * hh + g2 * htd + g3 * hrd)
    den6 = jnp.maximum(1.0 + 2.0 * xy6 + pp * ressq, EPS)
    f = 1.0 + 2.0 * xy6 + ressq
    ee = 1.0 - pp
    s1 = (-f * cp + ee * g1) / den6
    s2 = ee * g2 / den6
    s3c = ee * g3 / den6
    nsq = (s1 * s1 * hh + s2 * s2 * tt + s3c * s3c * rr
           + 2.0 * s1 * s2 * htd + 2.0 * s1 * s3c * hrd + 2.0 * s2 * s3c * trd)
    n6 = nrm(nsq)
    na = jnp.minimum(n6, 1.0 - EPS)
    at = 0.5 * _ln((1.0 + na) / (1.0 - na))
    kap = clam * at / n6
    return kap * s1, kap * s2, kap * s3c


NB = 5


def _make_sc_kernel(N, E, D, R):
    EW = E // (NC * NS)
    CH = EW // K
    BCH = CH // NB
    G = K // L
    NJ = D // L
    mesh = plsc.VectorSubcoreMesh(core_axis_name="c", subcore_axis_name="s")

    @functools.partial(
        pl.kernel,
        out_type=[jax.ShapeDtypeStruct((NC, N, D), jnp.float32),
                  jax.ShapeDtypeStruct((NC * N,), jnp.float32)],
        mesh=mesh,
        scratch_types=[
            pltpu.VMEM_SHARED((N, D), jnp.float32),
            pltpu.VMEM_SHARED((N,), jnp.float32),
            pltpu.VMEM((R, D), jnp.float32),
            pltpu.VMEM((CH // NB, K), jnp.int32),
            pltpu.VMEM((CH // NB, K), jnp.int32),
            pltpu.VMEM((CH // NB, K), jnp.int32),
            pltpu.VMEM((96,), jnp.int32),
            pltpu.VMEM((K,), jnp.int32),
            pltpu.VMEM((K,), jnp.int32),
            pltpu.VMEM((K, D), jnp.float32),
            pltpu.VMEM((K, D), jnp.float32),
            pltpu.VMEM((K, D), jnp.float32),
            pltpu.VMEM((K,), jnp.float32),
            pltpu.VMEM((K,), jnp.float32),
            pltpu.VMEM((K,), jnp.float32),
            pltpu.VMEM((K,), jnp.float32),
            pltpu.VMEM((K,), jnp.float32),
            pltpu.VMEM((96,), jnp.float32),
            pltpu.VMEM((96,), jnp.float32),
            pltpu.VMEM((96,), jnp.float32),
            pltpu.VMEM((K,), jnp.float32),
            pltpu.VMEM((L, 2 * L), jnp.float32),
            pltpu.VMEM((640,), jnp.float32),
            pltpu.VMEM((640,), jnp.float32),
            pltpu.SemaphoreType.DMA,
            pltpu.SemaphoreType.DMA,
            pltpu.SemaphoreType.DMA,
        ],
    )
    def agg(ego, head3, tail3, typ3, rel, sqtab, prtab, rsqtab,
            sums_out, cnt_out,
            acc, cacc, relv, hidx2, tidx2, ty2, ty96, pixh, pixt,
            hrows, trows, orows, sqh, sqt, prh, prt, rsqv,
            whb, wtb, wrb, onesv, pbuf, zcnt, cstage,
            sem_h, sem_t, sem_s):
        c = lax.axis_index("c")
        s = lax.axis_index("s")
        wid = s * NC + c
        iota = lax.iota(jnp.int32, L)
        zero16 = jnp.zeros((L,), jnp.float32)
        one16 = jnp.ones((L,), jnp.float32)

        pltpu.sync_copy(rel, relv)

        def zfill(e, _):
            for j in range(NJ):
                orows[e, pl.ds(j * L, L)] = zero16
            return 0
        lax.fori_loop(0, K, zfill, 0)
        for i in range(640 // L):
            zcnt[pl.ds(i * L, L)] = zero16
        for g in range(G):
            onesv[pl.ds(g * L, L)] = one16

        r0 = s * 624
        for b in range(640 // K):
            pltpu.sync_copy(orows, acc.at[pl.ds(r0 + b * K, K)])
        pltpu.sync_copy(zcnt, cacc.at[pl.ds(r0, 640)])
        plsc.subcore_barrier()

        def chunk(i, _):
            for g in range(G):
                sl = pl.ds(g * L, L)
                hv = hidx2[i, sl]
                tv = tidx2[i, sl]
                yv = ty2[i, sl]
                pixh[sl] = hv * R + yv
                pixt[sl] = tv * R + yv
                ty96[sl] = yv
            dh = pltpu.async_copy(ego.at[hidx2.at[i]], hrows, sem_h)
            dt = pltpu.async_copy(ego.at[tidx2.at[i]], trows, sem_t)
            g1 = pltpu.async_copy(sqtab.at[hidx2.at[i]], sqh, sem_s)
            g2 = pltpu.async_copy(sqtab.at[tidx2.at[i]], sqt, sem_s)
            g3 = pltpu.async_copy(prtab.at[pixh], prh, sem_s)
            g4 = pltpu.async_copy(prtab.at[pixt], prt, sem_s)
            g5 = pltpu.async_copy(rsqtab.at[ty2.at[i]], rsqv, sem_s)
            dh.wait()
            dt.wait()
            g1.wait()
            g2.wait()
            g3.wait()
            g4.wait()
            g5.wait()

            for g in range(G):
                sl = pl.ds(g * L, L)

                def partial(e2, _, g=g):
                    e = g * L + e2
                    pr = hrows[e, pl.ds(0, L)] * trows[e, pl.ds(0, L)]
                    for j in range(1, NJ):
                        pr = pr + hrows[e, pl.ds(j * L, L)] * trows[e, pl.ds(j * L, L)]
                    pbuf[e2, pl.ds(0, L)] = pr
                    return 0
                lax.fori_loop(0, L, partial, 0)

                for half in (8, 4, 2):
                    for e2 in range(L):
                        pbuf[e2, pl.ds(0, L)] = (pbuf[e2, pl.ds(0, L)]
                                                 + pbuf[e2, pl.ds(half, L)])
                htd = zero16
                for e2 in range(L):
                    v = pbuf[e2, pl.ds(0, L)] + pbuf[e2, pl.ds(1, L)]
                    htd = jnp.where(iota == e2, v[0], htd)

                wh, wt, wr = _edge_scalars(sqh[sl], sqt[sl], rsqv[sl],
                                           htd, prh[sl], prt[sl])
                whb[sl] = wh
                wtb[sl] = wt
                wrb[sl] = wr

            def combine(e, _):
                w1 = whb[pl.ds(e, L)][0]
                w2 = wtb[pl.ds(e, L)][0]
                w3 = wrb[pl.ds(e, L)][0]
                ty = ty96[pl.ds(e, L)][0]
                for j in range(NJ):
                    js = pl.ds(j * L, L)
                    orows[e, js] = (w1 * hrows[e, js] + w2 * trows[e, js]
                                    + w3 * relv[ty, js])
                return 0
            lax.fori_loop(0, K, combine, 0)

            pltpu.sync_copy(orows, acc.at[hidx2.at[i]], add=True)
            pltpu.sync_copy(onesv, cacc.at[hidx2.at[i]], add=True)
            return 0

        def block(bo, _):
            pltpu.sync_copy(head3.at[wid, bo], hidx2)
            pltpu.sync_copy(tail3.at[wid, bo], tidx2)
            pltpu.sync_copy(typ3.at[wid, bo], ty2)
            lax.fori_loop(0, BCH, chunk, 0)
            return 0
        lax.fori_loop(0, NB, block, 0)

        plsc.subcore_barrier()
        pltpu.sync_copy(acc.at[pl.ds(r0, 640)],
                        sums_out.at[c, pl.ds(r0, 640)])
        pltpu.sync_copy(cacc.at[pl.ds(r0, 640)], cstage)
        pltpu.sync_copy(cstage, cnt_out.at[pl.ds(c * N + r0, 640)])

    return agg


def _pre_body(ego_ref, rel_ref, sq_ref, pr_ref, rsq_ref):
    ego = ego_ref[...]
    rel = rel_ref[...]
    sq_ref[...] = jnp.sum(ego * ego, axis=1)
    rsq_ref[...] = jnp.sum(rel * rel, axis=1)
    pr_ref[...] = lax.dot_general(ego, rel, (((1,), (1,)), ((), ())),
                                  precision=lax.Precision.HIGHEST,
                                  preferred_element_type=jnp.float32)


def _combine_body(s_ref, c_ref, o_ref):
    tot = s_ref[0] + s_ref[1]
    cnt = jnp.maximum(c_ref[0] + c_ref[1], 1.0)
    o_ref[...] = tot / cnt[:, None]


@jax.jit
def kernel(ego_embed, edge_index, edge_type, relation_embed):
    N, D = ego_embed.shape
    E = edge_index.shape[1]
    R = relation_embed.shape[0]
    NW = NC * NS
    EW = E // NW
    CH = EW // K
    sq, pr, rsq = pl.pallas_call(
        _pre_body,
        out_shape=[jax.ShapeDtypeStruct((N,), jnp.float32),
                   jax.ShapeDtypeStruct((N, R), jnp.float32),
                   jax.ShapeDtypeStruct((R,), jnp.float32)],
    )(ego_embed, relation_embed)
    agg = _make_sc_kernel(N, E, D, R)
    head3 = edge_index[0].reshape(NW, NB, CH // NB, K)
    tail3 = edge_index[1].reshape(NW, NB, CH // NB, K)
    typ3 = edge_type.reshape(NW, NB, CH // NB, K)
    sums, cnts = agg(ego_embed, head3, tail3, typ3,
                     relation_embed, sq, pr.reshape(-1), rsq)
    out = pl.pallas_call(
        _combine_body,
        out_shape=jax.ShapeDtypeStruct((N, D), jnp.float32),
    )(sums, cnts.reshape(NC, N))
    return out


# speedup vs baseline: 4.2899x; 1.0055x over previous
import functools

import jax
import jax.numpy as jnp
from jax import lax
from jax.experimental import pallas as pl
from jax.experimental.pallas import tpu as pltpu
from jax.experimental.pallas import tpu_sc as plsc

EPS = 1e-7
MAXN = 1.0 - 1e-5
LN2 = 0.6931471805599453
L = 16
NC = 2
NS = 16
K = 80


def _rsqrt(x):
    i = lax.bitcast_convert_type(x, jnp.int32)
    i = 0x5F3759DF - lax.shift_right_logical(i, 1)
    y = lax.bitcast_convert_type(i, jnp.float32)
    for _ in range(3):
        y = y * (1.5 - 0.5 * x * y * y)
    return y


def _sqrt_pos(x):
    return x * _rsqrt(x)


def _tanh_pos(x):
    return 1.0 - 2.0 / (jnp.exp(2.0 * x) + 1.0)


def _ln(z):
    bits = lax.bitcast_convert_type(z, jnp.int32)
    e = lax.shift_right_logical(bits, 23) - 127
    mbits = jnp.bitwise_or(jnp.bitwise_and(bits, 0x007FFFFF), 0x3F800000)
    m = lax.bitcast_convert_type(mbits, jnp.float32)
    big = m > 1.4142135
    m = jnp.where(big, m * 0.5, m)
    e = (e + jnp.where(big, 1, 0)).astype(jnp.float32)
    s = (m - 1.0) / (m + 1.0)
    s2 = s * s
    lnm = 2.0 * s * (1.0 + s2 * (1.0 / 3.0 + s2 * (0.2 + s2 * (1.0 / 7.0 + s2 / 9.0))))
    return e * LN2 + lnm


def _edge_scalars(hh, tt, rr, htd, hrd, trd):
    def nrm(sq):
        return _sqrt_pos(jnp.clip(sq, EPS, None))

    def proj_scale(nsq):
        n = nrm(nsq)
        return jnp.where(n > MAXN, MAXN / n, 1.0)

    nh = nrm(hh)
    u = _tanh_pos(nh) / nh
    cp = u * proj_scale(u * u * hh)
    pp = cp * cp * hh
    clam = jnp.clip(1.0 - pp, EPS, None)
    lam = 2.0 / clam

    nt = nrm(tt)
    bt = _tanh_pos(0.5 * lam * nt) / nt
    y2 = bt * bt * tt
    xy = cp * bt * htd
    den = jnp.maximum(1.0 + 2.0 * xy + pp * y2, EPS)
    m1 = (1.0 + 2.0 * xy + y2) * cp / den
    m2 = (1.0 - pp) * bt / den
    vsq = m1 * m1 * hh + 2.0 * m1 * m2 * htd + m2 * m2 * tt
    s3 = proj_scale(vsq)
    a1 = m1 * s3
    a2 = m2 * s3
    htsq = s3 * s3 * vsq

    nr = nrm(rr)
    gm = _tanh_pos(0.5 * lam * nr) / nr
    y2r = gm * gm * rr
    xyr = cp * gm * hrd
    denr = jnp.maximum(1.0 + 2.0 * xyr + pp * y2r, EPS)
    k1 = (1.0 + 2.0 * xyr + y2r) * cp / denr
    k2 = (1.0 - pp) * gm / denr
    wsq = k1 * k1 * hh + 2.0 * k1 * k2 * hrd + k2 * k2 * rr
    s4 = proj_scale(wsq)
    b1 = k1 * s4
    b2 = k2 * s4
    hrsq = s4 * s4 * wsq

    xy5 = a1 * b1 * hh + a1 * b2 * hrd + a2 * b1 * htd + a2 * b2 * trd
    den5 = jnp.maximum(1.0 + 2.0 * xy5 + htsq * hrsq, EPS)
    cx = (1.0 + 2.0 * xy5 + hrsq) / den5
    cy = (1.0 - htsq) / den5
    g1 = cx * a1 + cy * b1
    g2 = cx * a2
    g3 = cy * b2
    vsq5 = (g1 * g1 * hh + g2 * g2 * tt + g3 * g3 * rr
            + 2.0 * g1 * g2 * htd + 2.0 * g1 * g3 * hrd + 2.0 * g2 * g3 * trd)
    s5 = proj_scale(vsq5)
    g1 = g1 * s5
    g2 = g2 * s5
    g3 = g3 * s5
    ressq = s5 * s5 * vsq5

    xy6 = -cp * (g1 * hh + g2 * htd + g3 * hrd)
    den6 = jnp.maximum(1.0 + 2.0 * xy6 + pp * ressq, EPS)
    f = 1.0 + 2.0 * xy6 + ressq
    ee = 1.0 - pp
    s1 = (-f * cp + ee * g1) / den6
    s2 = ee * g2 / den6
    s3c = ee * g3 / den6
    nsq = (s1 * s1 * hh + s2 * s2 * tt + s3c * s3c * rr
           + 2.0 * s1 * s2 * htd + 2.0 * s1 * s3c * hrd + 2.0 * s2 * s3c * trd)
    n6 = nrm(nsq)
    na = jnp.minimum(n6, 1.0 - EPS)
    at = 0.5 * _ln((1.0 + na) / (1.0 - na))
    kap = clam * at / n6
    return kap * s1, kap * s2, kap * s3c


NB = 5


def _make_sc_kernel(N, E, D, R):
    EW = E // (NC * NS)
    CH = EW // K
    BCH = CH // NB
    G = K // L
    NJ = D // L
    mesh = plsc.VectorSubcoreMesh(core_axis_name="c", subcore_axis_name="s")

    @functools.partial(
        pl.kernel,
        out_type=[jax.ShapeDtypeStruct((NC, N, D), jnp.float32),
                  jax.ShapeDtypeStruct((NC * N,), jnp.float32)],
        mesh=mesh,
        scratch_types=[
            pltpu.VMEM_SHARED((N, D), jnp.float32),
            pltpu.VMEM_SHARED((N,), jnp.float32),
            pltpu.VMEM((R, D), jnp.float32),
            pltpu.VMEM((CH // NB, K), jnp.int32),
            pltpu.VMEM((CH // NB, K), jnp.int32),
            pltpu.VMEM((CH // NB, K), jnp.int32),
            pltpu.VMEM((96,), jnp.int32),
            pltpu.VMEM((K,), jnp.int32),
            pltpu.VMEM((K,), jnp.int32),
            pltpu.VMEM((K, D), jnp.float32),
            pltpu.VMEM((K, D), jnp.float32),
            pltpu.VMEM((K, D), jnp.float32),
            pltpu.VMEM((K,), jnp.float32),
            pltpu.VMEM((K,), jnp.float32),
            pltpu.VMEM((K,), jnp.float32),
            pltpu.VMEM((K,), jnp.float32),
            pltpu.VMEM((K,), jnp.float32),
            pltpu.VMEM((96,), jnp.float32),
            pltpu.VMEM((96,), jnp.float32),
            pltpu.VMEM((96,), jnp.float32),
            pltpu.VMEM((K,), jnp.float32),
            pltpu.VMEM((L, 2 * L), jnp.float32),
            pltpu.VMEM((640,), jnp.float32),
            pltpu.VMEM((640,), jnp.float32),
            pltpu.SemaphoreType.DMA,
            pltpu.SemaphoreType.DMA,
            pltpu.SemaphoreType.DMA,
        ],
    )
    def agg(ego, head3, tail3, typ3, rel, sqtab, prtab, rsqtab,
            sums_out, cnt_out,
            acc, cacc, relv, hidx2, tidx2, ty2, ty96, pixh, pixt,
            hrows, trows, orows, sqh, sqt, prh, prt, rsqv,
            whb, wtb, wrb, onesv, pbuf, zcnt, cstage,
            sem_h, sem_t, sem_s):
        c = lax.axis_index("c")
        s = lax.axis_index("s")
        wid = s * NC + c
        iota = lax.iota(jnp.int32, L)
        zero16 = jnp.zeros((L,), jnp.float32)
        one16 = jnp.ones((L,), jnp.float32)

        pltpu.sync_copy(rel, relv)

        def zfill(e, _):
            for j in range(NJ):
                orows[e, pl.ds(j * L, L)] = zero16
            return 0
        lax.fori_loop(0, K, zfill, 0)
        for i in range(640 // L):
            zcnt[pl.ds(i * L, L)] = zero16
        for g in range(G):
            onesv[pl.ds(g * L, L)] = one16

        r0 = s * 624
        for b in range(640 // K):
            pltpu.sync_copy(orows, acc.at[pl.ds(r0 + b * K, K)])
        pltpu.sync_copy(zcnt, cacc.at[pl.ds(r0, 640)])
        plsc.subcore_barrier()

        def chunk(i, _):
            for g in range(G):
                sl = pl.ds(g * L, L)
                hv = hidx2[i, sl]
                tv = tidx2[i, sl]
                yv = ty2[i, sl]
                pixh[sl] = hv * R + yv
                pixt[sl] = tv * R + yv
                ty96[sl] = yv
            dh = pltpu.async_copy(ego.at[hidx2.at[i]], hrows, sem_h)
            dt = pltpu.async_copy(ego.at[tidx2.at[i]], trows, sem_t)
            g1 = pltpu.async_copy(sqtab.at[hidx2.at[i]], sqh, sem_s)
            g2 = pltpu.async_copy(sqtab.at[tidx2.at[i]], sqt, sem_s)
            g3 = pltpu.async_copy(prtab.at[pixh], prh, sem_s)
            g4 = pltpu.async_copy(prtab.at[pixt], prt, sem_s)
            g5 = pltpu.async_copy(rsqtab.at[ty2.at[i]], rsqv, sem_s)
            dh.wait()
            dt.wait()
            g1.wait()
            g2.wait()
            g3.wait()
            g4.wait()
            g5.wait()

            for g in range(G):
                sl = pl.ds(g * L, L)

                @plsc.parallel_loop(0, L, 1, unroll=2)
                def partial(e2, g=g):
                    e = g * L + e2
                    pr = hrows[e, pl.ds(0, L)] * trows[e, pl.ds(0, L)]
                    for j in range(1, NJ):
                        pr = pr + hrows[e, pl.ds(j * L, L)] * trows[e, pl.ds(j * L, L)]
                    pbuf[e2, pl.ds(0, L)] = pr

                for half in (8, 4, 2):
                    for e2 in range(L):
                        pbuf[e2, pl.ds(0, L)] = (pbuf[e2, pl.ds(0, L)]
                                                 + pbuf[e2, pl.ds(half, L)])
                htd = zero16
                for e2 in range(L):
                    v = pbuf[e2, pl.ds(0, L)] + pbuf[e2, pl.ds(1, L)]
                    htd = jnp.where(iota == e2, v[0], htd)

                wh, wt, wr = _edge_scalars(sqh[sl], sqt[sl], rsqv[sl],
                                           htd, prh[sl], prt[sl])
                whb[sl] = wh
                wtb[sl] = wt
                wrb[sl] = wr

            @plsc.parallel_loop(0, K, 1, unroll=2)
            def combine(e):
                w1 = whb[pl.ds(e, L)][0]
                w2 = wtb[pl.ds(e, L)][0]
                w3 = wrb[pl.ds(e, L)][0]
                ty = ty96[pl.ds(e, L)][0]
                for j in range(NJ):
                    js = pl.ds(j * L, L)
                    orows[e, js] = (w1 * hrows[e, js] + w2 * trows[e, js]
                                    + w3 * relv[ty, js])

            pltpu.sync_copy(orows, acc.at[hidx2.at[i]], add=True)
            pltpu.sync_copy(onesv, cacc.at[hidx2.at[i]], add=True)
            return 0

        def block(bo, _):
            pltpu.sync_copy(head3.at[wid, bo], hidx2)
            pltpu.sync_copy(tail3.at[wid, bo], tidx2)
            pltpu.sync_copy(typ3.at[wid, bo], ty2)
            lax.fori_loop(0, BCH, chunk, 0)
            return 0
        lax.fori_loop(0, NB, block, 0)

        plsc.subcore_barrier()
        pltpu.sync_copy(acc.at[pl.ds(r0, 640)],
                        sums_out.at[c, pl.ds(r0, 640)])
        pltpu.sync_copy(cacc.at[pl.ds(r0, 640)], cstage)
        pltpu.sync_copy(cstage, cnt_out.at[pl.ds(c * N + r0, 640)])

    return agg


def _pre_body(ego_ref, rel_ref, sq_ref, pr_ref, rsq_ref):
    ego = ego_ref[...]
    rel = rel_ref[...]
    sq_ref[...] = jnp.sum(ego * ego, axis=1)
    rsq_ref[...] = jnp.sum(rel * rel, axis=1)
    pr_ref[...] = lax.dot_general(ego, rel, (((1,), (1,)), ((), ())),
                                  precision=lax.Precision.HIGHEST,
                                  preferred_element_type=jnp.float32)


def _combine_body(s_ref, c_ref, o_ref):
    tot = s_ref[0] + s_ref[1]
    cnt = jnp.maximum(c_ref[0] + c_ref[1], 1.0)
    o_ref[...] = tot / cnt[:, None]


@jax.jit
def kernel(ego_embed, edge_index, edge_type, relation_embed):
    N, D = ego_embed.shape
    E = edge_index.shape[1]
    R = relation_embed.shape[0]
    NW = NC * NS
    EW = E // NW
    CH = EW // K
    sq, pr, rsq = pl.pallas_call(
        _pre_body,
        out_shape=[jax.ShapeDtypeStruct((N,), jnp.float32),
                   jax.ShapeDtypeStruct((N, R), jnp.float32),
                   jax.ShapeDtypeStruct((R,), jnp.float32)],
    )(ego_embed, relation_embed)
    agg = _make_sc_kernel(N, E, D, R)
    head3 = edge_index[0].reshape(NW, NB, CH // NB, K)
    tail3 = edge_index[1].reshape(NW, NB, CH // NB, K)
    typ3 = edge_type.reshape(NW, NB, CH // NB, K)
    sums, cnts = agg(ego_embed, head3, tail3, typ3,
                     relation_embed, sq, pr.reshape(-1), rsq)
    out = pl.pallas_call(
        _combine_body,
        out_shape=jax.ShapeDtypeStruct((N, D), jnp.float32),
    )(sums, cnts.reshape(NC, N))
    return out
